# Initial kernel scaffold; baseline (speedup 1.0000x reference)
import jax
import jax.numpy as jnp
from jax.experimental import pallas as pl


def kernel(coordinates, species, edge_index, shifts, batch):
    raise NotImplementedError("write your pallas kernel here")


# speedup vs baseline: 7.5026x; 7.5026x over previous
import functools

import jax
import jax.numpy as jnp
from jax import lax
from jax.experimental import pallas as pl
from jax.experimental.pallas import tpu as pltpu
from jax.experimental.pallas import tpu_sc as plsc

_NUM_CORES = 2
_NUM_SUBCORES = 16
_NW = _NUM_CORES * _NUM_SUBCORES
_LANES = 16
_GRP = 128
_CHUNK = 8
_CE = _CHUNK * _GRP

_C0, _C1, _C2, _C3 = 0.1818, 0.5099, 0.2802, 0.02817
_K0, _K1, _K2, _K3 = -3.2, -0.9423, -0.4028, -0.2016
_COUL = 14.3996
_INV_A_PRE = 1.0 / 0.4543


def _compute_chunk(gxs, gys, gzs, gws, gxd, gyd, gzd, gwd, sh, powv, en):
    lane = lax.iota(jnp.int32, _LANES)
    lane3 = lane * 3
    for r in range(_CHUNK):
        for j in range(_GRP // _LANES):
            co = _LANES * j
            sbase = lane3 + (3 * (_GRP * r + _LANES * j))
            xs = gxs[r, pl.ds(co, _LANES)]
            ys = gys[r, pl.ds(co, _LANES)]
            zs = gzs[r, pl.ds(co, _LANES)]
            zfs = gws[r, pl.ds(co, _LANES)]
            xd = gxd[r, pl.ds(co, _LANES)]
            yd = gyd[r, pl.ds(co, _LANES)]
            zd = gzd[r, pl.ds(co, _LANES)]
            zfd = gwd[r, pl.ds(co, _LANES)]
            hx = plsc.load_gather(sh, [sbase])
            hy = plsc.load_gather(sh, [sbase + 1])
            hz = plsc.load_gather(sh, [sbase + 2])
            zps = plsc.load_gather(powv, [zfs.astype(jnp.int32)])
            zpd = plsc.load_gather(powv, [zfd.astype(jnp.int32)])
            dx = xs - (xd + hx)
            dy = ys - (yd + hy)
            dz = zs - (zd + hz)
            s = dx * dx + dy * dy + dz * dz
            y = plsc.bitcast(
                jnp.int32(0x5F3759DF) - (plsc.bitcast(s, jnp.int32) >> 1),
                jnp.float32)
            y = y * (1.5 - 0.5 * s * y * y)
            y = y * (1.5 - 0.5 * s * y * y)
            y = y * (1.5 - 0.5 * s * y * y)
            d = s * y + 1e-9
            roa = d * (zps + zpd) * _INV_A_PRE
            phi = (_C0 * jnp.exp(_K0 * roa) + _C1 * jnp.exp(_K1 * roa)
                   + _C2 * jnp.exp(_K2 * roa) + _C3 * jnp.exp(_K3 * roa))
            en[r, pl.ds(co, _LANES)] = (_COUL * zfs * zfd) * phi / d


def _sc_body(rows_lo, extra_tiles,
             tx_hbm, ty_hbm, tz_hbm, tw_hbm, pow_hbm,
             esrc_hbm, edst_hbm, shf_hbm, zeros_hbm,
             out_hbm,
             tx_sh, ty_sh, tz_sh, tw_sh, acc_sh, powv,
             idx_s, idx_d, sh,
             gxs, gys, gzs, gws, gxd, gyd, gzd, gwd, en):
    c = lax.axis_index("c")
    s = lax.axis_index("s")
    wid = s * _NUM_CORES + c

    @pl.when(s == 0)
    def _stage():
        pltpu.sync_copy(tx_hbm, tx_sh)
        pltpu.sync_copy(ty_hbm, ty_sh)
        pltpu.sync_copy(tz_hbm, tz_sh)
        pltpu.sync_copy(tw_hbm, tw_sh)
        pltpu.sync_copy(zeros_hbm, acc_sh)

    pltpu.sync_copy(pow_hbm, powv)
    plsc.subcore_barrier()

    nex = jnp.minimum(wid, extra_tiles)
    row_base = wid * rows_lo + nex * _CHUNK
    n_chunks = rows_lo // _CHUNK + jnp.where(wid < extra_tiles, 1, 0)

    def chunk(cc, carry):
        r0 = pl.multiple_of(row_base + cc * _CHUNK, _CHUNK)
        pltpu.sync_copy(esrc_hbm.at[pl.ds(r0, _CHUNK)], idx_s)
        pltpu.sync_copy(edst_hbm.at[pl.ds(r0, _CHUNK)], idx_d)
        pltpu.sync_copy(shf_hbm.at[pl.ds(r0 * (3 * _GRP), 3 * _CE)], sh)
        for r in range(_CHUNK):
            irs = idx_s.at[r]
            ird = idx_d.at[r]
            pltpu.sync_copy(tx_sh.at[irs], gxs.at[r])
            pltpu.sync_copy(ty_sh.at[irs], gys.at[r])
            pltpu.sync_copy(tz_sh.at[irs], gzs.at[r])
            pltpu.sync_copy(tw_sh.at[irs], gws.at[r])
            pltpu.sync_copy(tx_sh.at[ird], gxd.at[r])
            pltpu.sync_copy(ty_sh.at[ird], gyd.at[r])
            pltpu.sync_copy(tz_sh.at[ird], gzd.at[r])
            pltpu.sync_copy(tw_sh.at[ird], gwd.at[r])
        _compute_chunk(gxs, gys, gzs, gws, gxd, gyd, gzd, gwd, sh, powv, en)
        for r in range(_CHUNK):
            pltpu.sync_copy(en.at[r], acc_sh.at[idx_s.at[r]], add=True)
        return carry

    lax.fori_loop(0, n_chunks, chunk, 0)

    plsc.subcore_barrier()

    @pl.when(s == 0)
    def _flush():
        pltpu.sync_copy(acc_sh, out_hbm.at[c])


def _tc_body(num_graphs, p_ref, b_ref, node_ref, graph_ref):
    node = p_ref[0] + p_ref[1]
    node_ref[...] = node
    bid = b_ref[...]
    for g in range(num_graphs):
        graph_ref[g] = jnp.sum(jnp.where(bid == g, node, 0.0))


def kernel(coordinates, species, edge_index, shifts, batch):
    n = coordinates.shape[0]
    e = edge_index.shape[1]
    num_graphs = 64
    assert e % _GRP == 0
    total_rows = e // _GRP
    rows_lo = (total_rows // _NW) // _CHUNK * _CHUNK
    extra = total_rows - rows_lo * _NW
    assert extra % _CHUNK == 0
    extra_tiles = extra // _CHUNK
    assert extra_tiles <= _NW

    tx = coordinates[:, 0]
    ty = coordinates[:, 1]
    tz = coordinates[:, 2]
    tw = species.astype(jnp.float32)
    powtab = jnp.power(jnp.arange(128, dtype=jnp.float32), jnp.float32(0.3))
    zeros_n = jnp.zeros((n,), jnp.float32)
    esrc = edge_index[0].reshape(total_rows, _GRP)
    edst = edge_index[1].reshape(total_rows, _GRP)
    shf = shifts.reshape(e * 3)

    mesh = plsc.VectorSubcoreMesh(core_axis_name="c", subcore_axis_name="s")
    sc_fn = pl.kernel(
        functools.partial(_sc_body, rows_lo, extra_tiles),
        out_type=jax.ShapeDtypeStruct((2, n), jnp.float32),
        mesh=mesh,
        compiler_params=pltpu.CompilerParams(use_tc_tiling_on_sc=False,
                                             needs_layout_passes=False),
        scratch_types=[
            pltpu.VMEM_SHARED((n,), jnp.float32),
            pltpu.VMEM_SHARED((n,), jnp.float32),
            pltpu.VMEM_SHARED((n,), jnp.float32),
            pltpu.VMEM_SHARED((n,), jnp.float32),
            pltpu.VMEM_SHARED((n,), jnp.float32),
            pltpu.VMEM((128,), jnp.float32),
            pltpu.VMEM((_CHUNK, _GRP), jnp.int32),
            pltpu.VMEM((_CHUNK, _GRP), jnp.int32),
            pltpu.VMEM((3 * _CE,), jnp.float32),
            pltpu.VMEM((_CHUNK, _GRP), jnp.float32),
            pltpu.VMEM((_CHUNK, _GRP), jnp.float32),
            pltpu.VMEM((_CHUNK, _GRP), jnp.float32),
            pltpu.VMEM((_CHUNK, _GRP), jnp.float32),
            pltpu.VMEM((_CHUNK, _GRP), jnp.float32),
            pltpu.VMEM((_CHUNK, _GRP), jnp.float32),
            pltpu.VMEM((_CHUNK, _GRP), jnp.float32),
            pltpu.VMEM((_CHUNK, _GRP), jnp.float32),
            pltpu.VMEM((_CHUNK, _GRP), jnp.float32),
        ],
    )
    partial = sc_fn(tx, ty, tz, tw, powtab, esrc, edst, shf, zeros_n)

    rows = -(-n // 128)
    np_ = rows * 128
    p_pad = jnp.pad(partial, ((0, 0), (0, np_ - n))).reshape(2, rows, 128)
    b_pad = jnp.pad(batch, (0, np_ - n)).reshape(rows, 128)

    node_pad, graph_energy = pl.pallas_call(
        functools.partial(_tc_body, num_graphs),
        out_shape=[
            jax.ShapeDtypeStruct((rows, 128), jnp.float32),
            jax.ShapeDtypeStruct((num_graphs,), jnp.float32),
        ],
        out_specs=[
            pl.BlockSpec(memory_space=pltpu.MemorySpace.VMEM),
            pl.BlockSpec(memory_space=pltpu.MemorySpace.SMEM),
        ],
    )(p_pad, b_pad)

    node_energy = node_pad.reshape(np_)[:n]
    return (node_energy, graph_energy)


# speedup vs baseline: 7.7811x; 1.0371x over previous
import functools

import jax
import jax.numpy as jnp
from jax import lax
from jax.experimental import pallas as pl
from jax.experimental.pallas import tpu as pltpu
from jax.experimental.pallas import tpu_sc as plsc

_NUM_CORES = 2
_NUM_SUBCORES = 16
_NW = _NUM_CORES * _NUM_SUBCORES
_LANES = 16
_GRP = 128
_CHUNK = 8
_CE = _CHUNK * _GRP

_C0, _C1, _C2, _C3 = 0.1818, 0.5099, 0.2802, 0.02817
_K0, _K1, _K2, _K3 = -3.2, -0.9423, -0.4028, -0.2016
_COUL = 14.3996
_INV_A_PRE = 1.0 / 0.4543


def _compute_chunk(gxs, gys, gzs, gws, gxd, gyd, gzd, gwd, sh, powv, en):
    lane = lax.iota(jnp.int32, _LANES)
    lane3 = lane * 3
    for r in range(_CHUNK):
        for j in range(_GRP // _LANES):
            co = _LANES * j
            sbase = lane3 + (3 * (_GRP * r + _LANES * j))
            xs = gxs[r, pl.ds(co, _LANES)]
            ys = gys[r, pl.ds(co, _LANES)]
            zs = gzs[r, pl.ds(co, _LANES)]
            zfs = gws[r, pl.ds(co, _LANES)]
            xd = gxd[r, pl.ds(co, _LANES)]
            yd = gyd[r, pl.ds(co, _LANES)]
            zd = gzd[r, pl.ds(co, _LANES)]
            zfd = gwd[r, pl.ds(co, _LANES)]
            hx = plsc.load_gather(sh, [sbase])
            hy = plsc.load_gather(sh, [sbase + 1])
            hz = plsc.load_gather(sh, [sbase + 2])
            zps = plsc.load_gather(powv, [zfs.astype(jnp.int32)])
            zpd = plsc.load_gather(powv, [zfd.astype(jnp.int32)])
            dx = xs - (xd + hx)
            dy = ys - (yd + hy)
            dz = zs - (zd + hz)
            s = dx * dx + dy * dy + dz * dz
            y = plsc.bitcast(
                jnp.int32(0x5F3759DF) - (plsc.bitcast(s, jnp.int32) >> 1),
                jnp.float32)
            y = y * (1.5 - 0.5 * s * y * y)
            y = y * (1.5 - 0.5 * s * y * y)
            y = y * (1.5 - 0.5 * s * y * y)
            d = s * y + 1e-9
            roa = d * (zps + zpd) * _INV_A_PRE
            phi = (_C0 * jnp.exp(_K0 * roa) + _C1 * jnp.exp(_K1 * roa)
                   + _C2 * jnp.exp(_K2 * roa) + _C3 * jnp.exp(_K3 * roa))
            en[r, pl.ds(co, _LANES)] = (_COUL * zfs * zfd) * phi / d


def _sc_body(rows_lo, extra_tiles,
             tx_hbm, ty_hbm, tz_hbm, tw_hbm, pow_hbm,
             esrc_hbm, edst_hbm, shf_hbm, zeros_hbm,
             out_hbm,
             tx_sh, ty_sh, tz_sh, tw_sh, acc_sh, powv,
             idx_s, idx_d, sh,
             gxs, gys, gzs, gws, gxd, gyd, gzd, gwd, en, semg, sems):
    c = lax.axis_index("c")
    s = lax.axis_index("s")
    wid = s * _NUM_CORES + c

    @pl.when(s == 0)
    def _stage():
        pltpu.sync_copy(tx_hbm, tx_sh)
        pltpu.sync_copy(ty_hbm, ty_sh)
        pltpu.sync_copy(tz_hbm, tz_sh)
        pltpu.sync_copy(tw_hbm, tw_sh)
        pltpu.sync_copy(zeros_hbm, acc_sh)

    pltpu.sync_copy(pow_hbm, powv)
    plsc.subcore_barrier()

    nex = jnp.minimum(wid, extra_tiles)
    row_base = wid * rows_lo + nex * _CHUNK
    n_chunks = rows_lo // _CHUNK + jnp.where(wid < extra_tiles, 1, 0)

    def chunk(cc, carry):
        r0 = pl.multiple_of(row_base + cc * _CHUNK, _CHUNK)
        dsh = pltpu.async_copy(
            shf_hbm.at[pl.ds(r0 * (3 * _GRP), 3 * _CE)], sh, semg)
        pltpu.sync_copy(esrc_hbm.at[pl.ds(r0, _CHUNK)], idx_s)
        pltpu.sync_copy(edst_hbm.at[pl.ds(r0, _CHUNK)], idx_d)
        descs = [dsh]
        for r in range(_CHUNK):
            irs = idx_s.at[r]
            ird = idx_d.at[r]
            descs.append(pltpu.async_copy(tx_sh.at[irs], gxs.at[r], semg))
            descs.append(pltpu.async_copy(ty_sh.at[irs], gys.at[r], semg))
            descs.append(pltpu.async_copy(tz_sh.at[irs], gzs.at[r], semg))
            descs.append(pltpu.async_copy(tw_sh.at[irs], gws.at[r], semg))
            descs.append(pltpu.async_copy(tx_sh.at[ird], gxd.at[r], semg))
            descs.append(pltpu.async_copy(ty_sh.at[ird], gyd.at[r], semg))
            descs.append(pltpu.async_copy(tz_sh.at[ird], gzd.at[r], semg))
            descs.append(pltpu.async_copy(tw_sh.at[ird], gwd.at[r], semg))
        for dsc in descs:
            dsc.wait()
        _compute_chunk(gxs, gys, gzs, gws, gxd, gyd, gzd, gwd, sh, powv, en)
        sdescs = []
        for r in range(_CHUNK):
            sdescs.append(pltpu.async_copy(
                en.at[r], acc_sh.at[idx_s.at[r]], sems, add=True))
        for dsc in sdescs:
            dsc.wait()
        return carry

    lax.fori_loop(0, n_chunks, chunk, 0)

    plsc.subcore_barrier()

    @pl.when(s == 0)
    def _flush():
        pltpu.sync_copy(acc_sh, out_hbm.at[c])


def _tc_body(num_graphs, p_ref, b_ref, node_ref, graph_ref):
    node = p_ref[0] + p_ref[1]
    node_ref[...] = node
    bid = b_ref[...]
    for g in range(num_graphs):
        graph_ref[g] = jnp.sum(jnp.where(bid == g, node, 0.0))


def kernel(coordinates, species, edge_index, shifts, batch):
    n = coordinates.shape[0]
    e = edge_index.shape[1]
    num_graphs = 64
    assert e % _GRP == 0
    total_rows = e // _GRP
    rows_lo = (total_rows // _NW) // _CHUNK * _CHUNK
    extra = total_rows - rows_lo * _NW
    assert extra % _CHUNK == 0
    extra_tiles = extra // _CHUNK
    assert extra_tiles <= _NW

    tx = coordinates[:, 0]
    ty = coordinates[:, 1]
    tz = coordinates[:, 2]
    tw = species.astype(jnp.float32)
    powtab = jnp.power(jnp.arange(128, dtype=jnp.float32), jnp.float32(0.3))
    zeros_n = jnp.zeros((n,), jnp.float32)
    esrc = edge_index[0].reshape(total_rows, _GRP)
    edst = edge_index[1].reshape(total_rows, _GRP)
    shf = shifts.reshape(e * 3)

    mesh = plsc.VectorSubcoreMesh(core_axis_name="c", subcore_axis_name="s")
    sc_fn = pl.kernel(
        functools.partial(_sc_body, rows_lo, extra_tiles),
        out_type=jax.ShapeDtypeStruct((2, n), jnp.float32),
        mesh=mesh,
        compiler_params=pltpu.CompilerParams(use_tc_tiling_on_sc=False,
                                             needs_layout_passes=False),
        scratch_types=[
            pltpu.VMEM_SHARED((n,), jnp.float32),
            pltpu.VMEM_SHARED((n,), jnp.float32),
            pltpu.VMEM_SHARED((n,), jnp.float32),
            pltpu.VMEM_SHARED((n,), jnp.float32),
            pltpu.VMEM_SHARED((n,), jnp.float32),
            pltpu.VMEM((128,), jnp.float32),
            pltpu.VMEM((_CHUNK, _GRP), jnp.int32),
            pltpu.VMEM((_CHUNK, _GRP), jnp.int32),
            pltpu.VMEM((3 * _CE,), jnp.float32),
            pltpu.VMEM((_CHUNK, _GRP), jnp.float32),
            pltpu.VMEM((_CHUNK, _GRP), jnp.float32),
            pltpu.VMEM((_CHUNK, _GRP), jnp.float32),
            pltpu.VMEM((_CHUNK, _GRP), jnp.float32),
            pltpu.VMEM((_CHUNK, _GRP), jnp.float32),
            pltpu.VMEM((_CHUNK, _GRP), jnp.float32),
            pltpu.VMEM((_CHUNK, _GRP), jnp.float32),
            pltpu.VMEM((_CHUNK, _GRP), jnp.float32),
            pltpu.VMEM((_CHUNK, _GRP), jnp.float32),
            pltpu.SemaphoreType.DMA,
            pltpu.SemaphoreType.DMA,
        ],
    )
    partial = sc_fn(tx, ty, tz, tw, powtab, esrc, edst, shf, zeros_n)

    rows = -(-n // 128)
    np_ = rows * 128
    p_pad = jnp.pad(partial, ((0, 0), (0, np_ - n))).reshape(2, rows, 128)
    b_pad = jnp.pad(batch, (0, np_ - n)).reshape(rows, 128)

    node_pad, graph_energy = pl.pallas_call(
        functools.partial(_tc_body, num_graphs),
        out_shape=[
            jax.ShapeDtypeStruct((rows, 128), jnp.float32),
            jax.ShapeDtypeStruct((num_graphs,), jnp.float32),
        ],
        out_specs=[
            pl.BlockSpec(memory_space=pltpu.MemorySpace.VMEM),
            pl.BlockSpec(memory_space=pltpu.MemorySpace.SMEM),
        ],
    )(p_pad, b_pad)

    node_energy = node_pad.reshape(np_)[:n]
    return (node_energy, graph_energy)


# speedup vs baseline: 7.8027x; 1.0028x over previous
import functools

import jax
import jax.numpy as jnp
from jax import lax
from jax.experimental import pallas as pl
from jax.experimental.pallas import tpu as pltpu
from jax.experimental.pallas import tpu_sc as plsc

_NUM_CORES = 2
_NUM_SUBCORES = 16
_NW = _NUM_CORES * _NUM_SUBCORES
_LANES = 16
_GRP = 128
_CHUNK = 8
_CE = _CHUNK * _GRP

_C0, _C1, _C2, _C3 = 0.1818, 0.5099, 0.2802, 0.02817
_K0, _K1, _K2, _K3 = -3.2, -0.9423, -0.4028, -0.2016
_COUL = 14.3996
_INV_A_PRE = 1.0 / 0.4543


def _compute_chunk(gxs, gys, gzs, gws, gxd, gyd, gzd, gwd, sh, powv, en):
    lane = lax.iota(jnp.int32, _LANES)
    lane3 = lane * 3
    for r in range(_CHUNK):
        for j in range(_GRP // _LANES):
            co = _LANES * j
            sbase = lane3 + (3 * (_GRP * r + _LANES * j))
            xs = gxs[r, pl.ds(co, _LANES)]
            ys = gys[r, pl.ds(co, _LANES)]
            zs = gzs[r, pl.ds(co, _LANES)]
            zfs = gws[r, pl.ds(co, _LANES)]
            xd = gxd[r, pl.ds(co, _LANES)]
            yd = gyd[r, pl.ds(co, _LANES)]
            zd = gzd[r, pl.ds(co, _LANES)]
            zfd = gwd[r, pl.ds(co, _LANES)]
            hx = plsc.load_gather(sh, [sbase])
            hy = plsc.load_gather(sh, [sbase + 1])
            hz = plsc.load_gather(sh, [sbase + 2])
            zps = plsc.load_gather(powv, [zfs.astype(jnp.int32)])
            zpd = plsc.load_gather(powv, [zfd.astype(jnp.int32)])
            dx = xs - (xd + hx)
            dy = ys - (yd + hy)
            dz = zs - (zd + hz)
            s = dx * dx + dy * dy + dz * dz
            y = plsc.bitcast(
                jnp.int32(0x5F3759DF) - (plsc.bitcast(s, jnp.int32) >> 1),
                jnp.float32)
            y = y * (1.5 - 0.5 * s * y * y)
            y = y * (1.5 - 0.5 * s * y * y)
            y = y * (1.5 - 0.5 * s * y * y)
            d = s * y + 1e-9
            roa = d * (zps + zpd) * _INV_A_PRE
            phi = (_C0 * jnp.exp(_K0 * roa) + _C1 * jnp.exp(_K1 * roa)
                   + _C2 * jnp.exp(_K2 * roa) + _C3 * jnp.exp(_K3 * roa))
            en[r, pl.ds(co, _LANES)] = (_COUL * zfs * zfd) * phi / d


def _sc_body(rows_lo, extra_tiles,
             tx_hbm, ty_hbm, tz_hbm, tw_hbm, pow_hbm,
             esrc_hbm, edst_hbm, shf_hbm, zeros_hbm,
             out_hbm,
             tx_sh, ty_sh, tz_sh, tw_sh, acc_sh, powv,
             idx_s, idx_d, sh,
             gxs, gys, gzs, gws, gxd, gyd, gzd, gwd, en, semg, sems, semi):
    c = lax.axis_index("c")
    s = lax.axis_index("s")
    wid = s * _NUM_CORES + c

    @pl.when(s == 0)
    def _stage():
        pltpu.sync_copy(tx_hbm, tx_sh)
        pltpu.sync_copy(ty_hbm, ty_sh)
        pltpu.sync_copy(tz_hbm, tz_sh)
        pltpu.sync_copy(tw_hbm, tw_sh)
        pltpu.sync_copy(zeros_hbm, acc_sh)

    pltpu.sync_copy(pow_hbm, powv)
    plsc.subcore_barrier()

    nex = jnp.minimum(wid, extra_tiles)
    row_base = wid * rows_lo + nex * _CHUNK
    n_chunks = rows_lo // _CHUNK + jnp.where(wid < extra_tiles, 1, 0)

    def chunk(cc, carry):
        r0 = pl.multiple_of(row_base + cc * _CHUNK, _CHUNK)
        dsh = pltpu.async_copy(
            shf_hbm.at[pl.ds(r0 * (3 * _GRP), 3 * _CE)], sh, semg)
        b0 = r0 * _GRP
        idescs = []
        for r in range(_CHUNK):
            idescs.append(pltpu.async_copy(
                esrc_hbm.at[pl.ds(b0 + _GRP * r, _GRP)], idx_s.at[r], semi))
            idescs.append(pltpu.async_copy(
                edst_hbm.at[pl.ds(b0 + _GRP * r, _GRP)], idx_d.at[r], semi))
        for dsc in idescs:
            dsc.wait()
        descs = [dsh]
        for r in range(_CHUNK):
            irs = idx_s.at[r]
            ird = idx_d.at[r]
            descs.append(pltpu.async_copy(tx_sh.at[irs], gxs.at[r], semg))
            descs.append(pltpu.async_copy(ty_sh.at[irs], gys.at[r], semg))
            descs.append(pltpu.async_copy(tz_sh.at[irs], gzs.at[r], semg))
            descs.append(pltpu.async_copy(tw_sh.at[irs], gws.at[r], semg))
            descs.append(pltpu.async_copy(tx_sh.at[ird], gxd.at[r], semg))
            descs.append(pltpu.async_copy(ty_sh.at[ird], gyd.at[r], semg))
            descs.append(pltpu.async_copy(tz_sh.at[ird], gzd.at[r], semg))
            descs.append(pltpu.async_copy(tw_sh.at[ird], gwd.at[r], semg))
        for dsc in descs:
            dsc.wait()
        _compute_chunk(gxs, gys, gzs, gws, gxd, gyd, gzd, gwd, sh, powv, en)
        sdescs = []
        for r in range(_CHUNK):
            sdescs.append(pltpu.async_copy(
                en.at[r], acc_sh.at[idx_s.at[r]], sems, add=True))
        for dsc in sdescs:
            dsc.wait()
        return carry

    lax.fori_loop(0, n_chunks, chunk, 0)

    plsc.subcore_barrier()

    @pl.when(s == 0)
    def _flush():
        n_nodes = acc_sh.shape[0]
        pltpu.sync_copy(acc_sh, out_hbm.at[pl.ds(c * n_nodes, n_nodes)])


def _tc_body(num_graphs, p_ref, b_ref, node_ref, graph_ref):
    node = p_ref[0] + p_ref[1]
    node_ref[...] = node
    bid = b_ref[...]
    for g in range(num_graphs):
        graph_ref[g] = jnp.sum(jnp.where(bid == g, node, 0.0))


def kernel(coordinates, species, edge_index, shifts, batch):
    n = coordinates.shape[0]
    e = edge_index.shape[1]
    num_graphs = 64
    assert e % _GRP == 0
    total_rows = e // _GRP
    rows_lo = (total_rows // _NW) // _CHUNK * _CHUNK
    extra = total_rows - rows_lo * _NW
    assert extra % _CHUNK == 0
    extra_tiles = extra // _CHUNK
    assert extra_tiles <= _NW

    tx = coordinates[:, 0]
    ty = coordinates[:, 1]
    tz = coordinates[:, 2]
    tw = species.astype(jnp.float32)
    powtab = jnp.power(jnp.arange(128, dtype=jnp.float32), jnp.float32(0.3))
    zeros_n = jnp.zeros((n,), jnp.float32)
    esrc = edge_index[0]
    edst = edge_index[1]
    shf = shifts.reshape(e * 3)

    mesh = plsc.VectorSubcoreMesh(core_axis_name="c", subcore_axis_name="s")
    sc_fn = pl.kernel(
        functools.partial(_sc_body, rows_lo, extra_tiles),
        out_type=jax.ShapeDtypeStruct((2 * n,), jnp.float32),
        mesh=mesh,
        compiler_params=pltpu.CompilerParams(use_tc_tiling_on_sc=False,
                                             needs_layout_passes=False),
        scratch_types=[
            pltpu.VMEM_SHARED((n,), jnp.float32),
            pltpu.VMEM_SHARED((n,), jnp.float32),
            pltpu.VMEM_SHARED((n,), jnp.float32),
            pltpu.VMEM_SHARED((n,), jnp.float32),
            pltpu.VMEM_SHARED((n,), jnp.float32),
            pltpu.VMEM((128,), jnp.float32),
            pltpu.VMEM((_CHUNK, _GRP), jnp.int32),
            pltpu.VMEM((_CHUNK, _GRP), jnp.int32),
            pltpu.VMEM((3 * _CE,), jnp.float32),
            pltpu.VMEM((_CHUNK, _GRP), jnp.float32),
            pltpu.VMEM((_CHUNK, _GRP), jnp.float32),
            pltpu.VMEM((_CHUNK, _GRP), jnp.float32),
            pltpu.VMEM((_CHUNK, _GRP), jnp.float32),
            pltpu.VMEM((_CHUNK, _GRP), jnp.float32),
            pltpu.VMEM((_CHUNK, _GRP), jnp.float32),
            pltpu.VMEM((_CHUNK, _GRP), jnp.float32),
            pltpu.VMEM((_CHUNK, _GRP), jnp.float32),
            pltpu.VMEM((_CHUNK, _GRP), jnp.float32),
            pltpu.SemaphoreType.DMA,
            pltpu.SemaphoreType.DMA,
            pltpu.SemaphoreType.DMA,
        ],
    )
    partial = sc_fn(tx, ty, tz, tw, powtab, esrc, edst, shf,
                    zeros_n).reshape(2, n)

    rows = -(-n // 128)
    np_ = rows * 128
    p_pad = jnp.pad(partial, ((0, 0), (0, np_ - n))).reshape(2, rows, 128)
    b_pad = jnp.pad(batch, (0, np_ - n)).reshape(rows, 128)

    node_pad, graph_energy = pl.pallas_call(
        functools.partial(_tc_body, num_graphs),
        out_shape=[
            jax.ShapeDtypeStruct((rows, 128), jnp.float32),
            jax.ShapeDtypeStruct((num_graphs,), jnp.float32),
        ],
        out_specs=[
            pl.BlockSpec(memory_space=pltpu.MemorySpace.VMEM),
            pl.BlockSpec(memory_space=pltpu.MemorySpace.SMEM),
        ],
    )(p_pad, b_pad)

    node_energy = node_pad.reshape(np_)[:n]
    return (node_energy, graph_energy)
